# baseline (speedup 1.0000x reference)
import functools

import jax
import jax.numpy as jnp
import numpy as np
from jax.experimental import pallas as pl
from jax.experimental.pallas import tpu as pltpu


def _hardswish(x):
    return x * jnp.clip(x + 3.0, 0.0, 6.0) * (1.0 / 6.0)


def _kernel_chunked(x_ref, bd_ref, dwb_ref, pw_ref, pwb_ref, o_ref, xp_ref):
    Nb, H, WC = x_ref.shape
    n_blocks = bd_ref.shape[0]
    nch = WC // 256
    f32 = jnp.float32
    G = 2 if Nb % 2 == 0 else 1
    Ng = Nb // G
    Rg = Ng * H

    xp_ref[:, 0:1, :] = jnp.zeros((Nb, 1, WC), f32)
    xp_ref[:, H + 1:H + 2, :] = jnp.zeros((Nb, 1, WC), f32)

    xs = [
        x_ref[g * Ng:(g + 1) * Ng].astype(f32).reshape(Rg, WC)
        for g in range(G)
    ]

    for blk in range(n_blocks):
        for g in range(G):
            xp_ref[g * Ng:(g + 1) * Ng, 1:H + 1, :] = xs[g].reshape(Ng, H, WC)
        for g in range(G):
            xp = xp_ref[g * Ng:(g + 1) * Ng]
            x = xs[g]

            accs = [None] * nch
            corners = [None] * (nch - 1)
            for kh in range(3):
                for j in range(nch):
                    s = xp[:, kh:kh + H, 256 * j:256 * (j + 1)].reshape(Rg, 256)
                    d = jnp.dot(s, bd_ref[blk, kh, j], preferred_element_type=f32)
                    accs[j] = d if accs[j] is None else accs[j] + d
                for j in range(nch - 1):
                    s = xp[:, kh:kh + H, 256 * j + 128:256 * j + 384].reshape(Rg, 256)
                    d = jnp.dot(s, bd_ref[blk, kh, nch + j],
                                preferred_element_type=f32)
                    corners[j] = d if corners[j] is None else corners[j] + d
            pieces = [accs[0][:, 0:128]]
            for j in range(nch - 1):
                pieces.append(accs[j][:, 128:256] + corners[j][:, 0:128])
                pieces.append(accs[j + 1][:, 0:128] + corners[j][:, 128:256])
            pieces.append(accs[nch - 1][:, 128:256])
            y = jnp.concatenate(pieces, axis=1)
            y = _hardswish(y + dwb_ref[blk].reshape(1, WC))

            z = jnp.concatenate(
                [
                    jnp.dot(y[:, 256 * j:256 * (j + 1)], pw_ref[blk],
                            preferred_element_type=f32)
                    for j in range(nch)
                ],
                axis=1,
            )
            z = _hardswish(z + pwb_ref[blk].reshape(1, WC))

            xs[g] = z + x

    for g in range(G):
        o_ref[g * Ng:(g + 1) * Ng] = (
            xs[g].reshape(Ng, H, WC).astype(o_ref.dtype))


def _kernel_full(n_pw_chunks, x_ref, bd_ref, dwb_ref, pw_ref, pwb_ref, o_ref,
                 xp_ref):
    Nb, H, WC = x_ref.shape
    n_blocks = bd_ref.shape[0]
    CH = pw_ref.shape[-1]
    R = Nb * H
    f32 = jnp.float32

    xp_ref[:, 0:1, :] = jnp.zeros((Nb, 1, WC), f32)
    xp_ref[:, H + 1:H + 2, :] = jnp.zeros((Nb, 1, WC), f32)

    x = x_ref[...].astype(f32).reshape(R, WC)

    for blk in range(n_blocks):
        xp_ref[:, 1:H + 1, :] = x.reshape(Nb, H, WC)
        a = xp_ref[:, 0:H, :].reshape(R, WC)
        c = xp_ref[:, 2:H + 2, :].reshape(R, WC)
        y = (jnp.dot(a, bd_ref[blk, 0], preferred_element_type=f32)
             + jnp.dot(x, bd_ref[blk, 1], preferred_element_type=f32)
             + jnp.dot(c, bd_ref[blk, 2], preferred_element_type=f32))
        y = _hardswish(y + dwb_ref[blk].reshape(1, WC))
        if n_pw_chunks == 1:
            z = jnp.dot(y, pw_ref[blk], preferred_element_type=f32)
        else:
            z = jnp.concatenate(
                [
                    jnp.dot(y[:, k * CH:(k + 1) * CH], pw_ref[blk],
                            preferred_element_type=f32)
                    for k in range(n_pw_chunks)
                ],
                axis=1,
            )
        z = _hardswish(z + pwb_ref[blk].reshape(1, WC))
        x = z + x

    o_ref[...] = x.reshape(Nb, H, WC).astype(o_ref.dtype)


def _band_masks(W, C):
    WC = W * C
    masks = np.zeros((3, WC, WC), np.float32)
    eye_c = np.eye(C, dtype=np.float32)
    for kw in range(3):
        for w in range(W):
            v = w + kw - 1
            if 0 <= v < W:
                masks[kw, v * C:(v + 1) * C, w * C:(w + 1) * C] = eye_c
    return masks


def _pw_mask(reps, C):
    m = np.zeros((reps * C, reps * C), np.float32)
    for u in range(reps):
        m[u * C:(u + 1) * C, u * C:(u + 1) * C] = 1.0
    return m


def _build_params(dww, dwb, pww, pwb, W):
    n, _, _, C = dww.shape
    WC = W * C
    dww_f = jnp.tile(dww[:, :, :, None, :], (1, 1, 1, W, 1)).reshape(n, 3, 3, WC)
    masks = _band_masks(W, C)
    chunked = WC % 256 == 0 and WC >= 512 and C <= 128 and 256 % C == 0
    if chunked:
        nch = WC // 256
        mid = np.arange(256)
        straddle = ((mid[:, None] < 128) != (mid[None, :] < 128)).astype(np.float32)
        mats = []
        for j in range(nch):
            sl = slice(256 * j, 256 * (j + 1))
            mats.append(sum(
                masks[kw][sl, sl][None, None] * dww_f[:, :, kw, None, sl]
                for kw in range(3)
            ))
        for j in range(nch - 1):
            sl = slice(256 * j + 128, 256 * j + 384)
            mats.append(sum(
                (masks[kw][sl, sl] * straddle)[None, None]
                * dww_f[:, :, kw, None, sl]
                for kw in range(3)
            ))
        bd = jnp.stack(mats, axis=2)
    else:
        bd = sum(
            masks[kw][None, None] * dww_f[:, :, kw, None, :]
            for kw in range(3)
        )
    dwb_f = jnp.tile(dwb, (1, W))
    pwb_f = jnp.tile(pwb, (1, W))
    ch = 256 if (WC % 256 == 0 and 256 % C == 0) else WC
    reps = ch // C
    pw_c = _pw_mask(reps, C) * jnp.tile(pww, (1, reps, reps))
    return bd, dwb_f, pw_c, pwb_f, chunked


def kernel(x_nhwc, dww, dwb, pww, pwb):
    N, H, W, C = x_nhwc.shape
    WC = W * C

    bd, dwb_f, pw_c, pwb_f, chunked = _build_params(dww, dwb, pww, pwb, W)
    n = bd.shape[0]
    ch = pw_c.shape[-1]
    x_f = x_nhwc.reshape(N, H, WC)

    Nb = next(nb for nb in (16, 8, 4, 2, 1) if N % nb == 0)

    if chunked:
        body = _kernel_chunked
    else:
        body = functools.partial(_kernel_full, WC // ch)
    nd = bd.ndim

    out = pl.pallas_call(
        body,
        out_shape=jax.ShapeDtypeStruct((N, H, WC), x_nhwc.dtype),
        grid_spec=pltpu.PrefetchScalarGridSpec(
            num_scalar_prefetch=0,
            grid=(N // Nb,),
            in_specs=[
                pl.BlockSpec((Nb, H, WC), lambda b: (b, 0, 0)),
                pl.BlockSpec(bd.shape, lambda b: (0,) * nd),
                pl.BlockSpec((n, WC), lambda b: (0, 0)),
                pl.BlockSpec((n, ch, ch), lambda b: (0, 0, 0)),
                pl.BlockSpec((n, WC), lambda b: (0, 0)),
            ],
            out_specs=pl.BlockSpec((Nb, H, WC), lambda b: (b, 0, 0)),
            scratch_shapes=[pltpu.VMEM((Nb, H + 2, WC), jnp.float32)],
        ),
        compiler_params=pltpu.CompilerParams(
            dimension_semantics=("parallel",),
            vmem_limit_bytes=64 * 1024 * 1024,
        ),
    )(x_f, bd, dwb_f, pw_c, pwb_f)
    return out.reshape(N, H, W, C)


# speedup vs baseline: 1.0896x; 1.0896x over previous
import functools

import jax
import jax.numpy as jnp
import numpy as np
from jax.experimental import pallas as pl
from jax.experimental.pallas import tpu as pltpu


def _hardswish(x):
    return x * jnp.clip(x + 3.0, 0.0, 6.0) * (1.0 / 6.0)


def _kernel(n_pw_chunks, x_ref, bd_ref, dwb_ref, pw_ref, pwb_ref, o_ref,
            xp_ref):
    Nb, H, WC = x_ref.shape
    n_blocks = bd_ref.shape[0]
    CH = pw_ref.shape[-1]
    R = Nb * H
    f32 = jnp.float32

    xp_ref[:, 0:1, :] = jnp.zeros((Nb, 1, WC), f32)
    xp_ref[:, H + 1:H + 2, :] = jnp.zeros((Nb, 1, WC), f32)

    x = x_ref[...].astype(f32).reshape(R, WC)

    for blk in range(n_blocks):
        xp_ref[:, 1:H + 1, :] = x.reshape(Nb, H, WC)

        a = xp_ref[:, 0:H, :].reshape(R, WC)
        c = xp_ref[:, 2:H + 2, :].reshape(R, WC)
        y = (jnp.dot(a, bd_ref[blk, 0], preferred_element_type=f32)
             + jnp.dot(x, bd_ref[blk, 1], preferred_element_type=f32)
             + jnp.dot(c, bd_ref[blk, 2], preferred_element_type=f32))
        y = _hardswish(y + dwb_ref[blk].reshape(1, WC))

        if n_pw_chunks == 1:
            z = jnp.dot(y, pw_ref[blk], preferred_element_type=f32)
        else:
            z = jnp.concatenate(
                [
                    jnp.dot(y[:, k * CH:(k + 1) * CH], pw_ref[blk],
                            preferred_element_type=f32)
                    for k in range(n_pw_chunks)
                ],
                axis=1,
            )
        z = _hardswish(z + pwb_ref[blk].reshape(1, WC))

        x = z + x

    o_ref[...] = x.reshape(Nb, H, WC).astype(o_ref.dtype)


def _band_masks(W, C):
    WC = W * C
    masks = np.zeros((3, WC, WC), np.float32)
    eye_c = np.eye(C, dtype=np.float32)
    for kw in range(3):
        for w in range(W):
            v = w + kw - 1
            if 0 <= v < W:
                masks[kw, v * C:(v + 1) * C, w * C:(w + 1) * C] = eye_c
    return masks


def _pw_mask(reps, C):
    m = np.zeros((reps * C, reps * C), np.float32)
    for u in range(reps):
        m[u * C:(u + 1) * C, u * C:(u + 1) * C] = 1.0
    return m


def _build_params(dww, dwb, pww, pwb, W):
    n, _, _, C = dww.shape
    WC = W * C
    dww_f = jnp.tile(dww[:, :, :, None, :], (1, 1, 1, W, 1)).reshape(n, 3, 3, WC)
    masks = _band_masks(W, C)
    bd = sum(
        masks[kw][None, None] * dww_f[:, :, kw, None, :]
        for kw in range(3)
    )
    dwb_f = jnp.tile(dwb, (1, W))
    pwb_f = jnp.tile(pwb, (1, W))
    ch = 256 if (WC % 256 == 0 and 256 % C == 0) else WC
    reps = ch // C
    pw_c = _pw_mask(reps, C) * jnp.tile(pww, (1, reps, reps))
    return bd, dwb_f, pw_c, pwb_f


def kernel(x_nhwc, dww, dwb, pww, pwb):
    N, H, W, C = x_nhwc.shape
    WC = W * C

    bd, dwb_f, pw_c, pwb_f = _build_params(dww, dwb, pww, pwb, W)
    n = bd.shape[0]
    ch = pw_c.shape[-1]
    x_f = x_nhwc.reshape(N, H, WC)

    Nb = next(nb for nb in (16, 8, 4, 2, 1) if N % nb == 0)

    out = pl.pallas_call(
        functools.partial(_kernel, WC // ch),
        out_shape=jax.ShapeDtypeStruct((N, H, WC), x_nhwc.dtype),
        grid_spec=pltpu.PrefetchScalarGridSpec(
            num_scalar_prefetch=0,
            grid=(N // Nb,),
            in_specs=[
                pl.BlockSpec((Nb, H, WC), lambda b: (b, 0, 0)),
                pl.BlockSpec((n, 3, WC, WC), lambda b: (0, 0, 0, 0)),
                pl.BlockSpec((n, WC), lambda b: (0, 0)),
                pl.BlockSpec((n, ch, ch), lambda b: (0, 0, 0)),
                pl.BlockSpec((n, WC), lambda b: (0, 0)),
            ],
            out_specs=pl.BlockSpec((Nb, H, WC), lambda b: (b, 0, 0)),
            scratch_shapes=[pltpu.VMEM((Nb, H + 2, WC), jnp.float32)],
        ),
        compiler_params=pltpu.CompilerParams(
            dimension_semantics=("parallel",),
            vmem_limit_bytes=64 * 1024 * 1024,
        ),
    )(x_f, bd, dwb_f, pw_c, pwb_f)
    return out.reshape(N, H, W, C)
